# baseline (speedup 1.0000x reference)
import functools

import jax
import jax.numpy as jnp
from jax import lax
from jax.experimental import pallas as pl
from jax.experimental.pallas import tpu as pltpu
from jax.experimental.pallas import tpu_sc as plsc

F32 = jnp.float32
I32 = jnp.int32

NC = 2
NS = 16
NT = NC * NS
CAP = 6144
KG = 48


def _mesh():
  return plsc.VectorSubcoreMesh(core_axis_name="c", subcore_axis_name="s")


def _wid():
  return lax.axis_index("s") * NC + lax.axis_index("c")


def _route_edges(src, dst, ew, n_nodes, rpt):
  E = src.shape[0]
  chunk = 2000
  while E % chunk:
    chunk //= 2
  n_chunks = E // chunk
  iters = chunk // 16

  def body(src_hbm, dst_hbm, ew_hbm, src_s, dstloc_s, norm_s, counts,
           srcch, dstch, ewch, srcb, dstb, ewb, deg, cnt_v):
    wid = _wid()
    lo = wid * rpt

    def chunk_body(k, pos):
      off = k * chunk
      pltpu.sync_copy(src_hbm.at[pl.ds(off, chunk)], srcch)
      pltpu.sync_copy(dst_hbm.at[pl.ds(off, chunk)], dstch)
      pltpu.sync_copy(ew_hbm.at[pl.ds(off, chunk)], ewch)

      def inner(i, pos):
        s16 = srcch[pl.ds(i * 16, 16)]
        d16 = dstch[pl.ds(i * 16, 16)]
        e16 = ewch[pl.ds(i * 16, 16)]
        dl = d16 - lo
        mask = (dl >= 0) & (dl < rpt)
        cnt = plsc.all_reduce_population_count(mask)[0]
        plsc.store_compressed(srcb.at[pl.ds(pos, 16)], s16, mask=mask)
        plsc.store_compressed(dstb.at[pl.ds(pos, 16)], dl, mask=mask)
        plsc.store_compressed(ewb.at[pl.ds(pos, 16)], e16, mask=mask)
        return pos + cnt

      return lax.fori_loop(0, iters, inner, pos)

    pos = lax.fori_loop(0, n_chunks, chunk_body, jnp.int32(0))

    z16i = jnp.zeros((16,), I32)
    z16f = jnp.zeros((16,), F32)
    for t in range(4):
      srcb[pl.ds(pos + t * 16, 16)] = z16i
      dstb[pl.ds(pos + t * 16, 16)] = z16i
      ewb[pl.ds(pos + t * 16, 16)] = z16f
    cnt_p = ((pos + KG - 1) // KG) * KG

    for r in range(rpt // 16):
      deg[pl.ds(r * 16, 16)] = z16f

    def deg_body(i, _):
      d16 = dstb[pl.ds(i * 16, 16)]
      e16 = ewb[pl.ds(i * 16, 16)]
      plsc.addupdate_scatter(deg, [d16], e16)
      return 0

    lax.fori_loop(0, cnt_p // 16, deg_body, 0)

    def norm_body(i, _):
      d16 = dstb[pl.ds(i * 16, 16)]
      e16 = ewb[pl.ds(i * 16, 16)]
      dg = plsc.load_gather(deg, [d16])
      ewb[pl.ds(i * 16, 16)] = e16 / jnp.maximum(dg, 1e-6)
      return 0

    lax.fori_loop(0, cnt_p // 16, norm_body, 0)

    pltpu.sync_copy(srcb.at[pl.ds(0, CAP)], src_s.at[wid])
    pltpu.sync_copy(dstb.at[pl.ds(0, CAP)], dstloc_s.at[wid])
    pltpu.sync_copy(ewb.at[pl.ds(0, CAP)], norm_s.at[wid])
    cnt_v[...] = jnp.full((16,), cnt_p, I32)
    pltpu.sync_copy(cnt_v, counts.at[wid])

  run = pl.kernel(
      body,
      out_type=(
          jax.ShapeDtypeStruct((NT, CAP), I32),
          jax.ShapeDtypeStruct((NT, CAP), I32),
          jax.ShapeDtypeStruct((NT, CAP), F32),
          jax.ShapeDtypeStruct((NT, 16), I32),
      ),
      mesh=_mesh(),
      compiler_params=pltpu.CompilerParams(needs_layout_passes=False),
      scratch_types=[
          pltpu.VMEM((chunk,), I32),
          pltpu.VMEM((chunk,), I32),
          pltpu.VMEM((chunk,), F32),
          pltpu.VMEM((CAP + 64,), I32),
          pltpu.VMEM((CAP + 64,), I32),
          pltpu.VMEM((CAP + 64,), F32),
          pltpu.VMEM((rpt,), F32),
          pltpu.VMEM((16,), I32),
      ],
  )
  return run(src, dst, ew)


def _aggregate(h, src_s, dstloc_s, norm_s, counts, np_, rpt):
  acc_n = rpt * 256

  def body(h_hbm, src_s_h, dstloc_s_h, norm_s_h, counts_h, out_hbm,
           src_v, dstloc_v, norm_v, rows_a, rows_b, out_acc, cnt_v,
           sem_a, sem_b):
    wid = _wid()
    pltpu.sync_copy(src_s_h.at[wid], src_v)
    pltpu.sync_copy(dstloc_s_h.at[wid], dstloc_v)
    pltpu.sync_copy(norm_s_h.at[wid], norm_v)
    pltpu.sync_copy(counts_h.at[wid], cnt_v)
    cnt_p = cnt_v[pl.ds(0, 16)][0]
    nch = cnt_p // KG

    z16 = jnp.zeros((16,), F32)

    @plsc.parallel_loop(0, rpt, 1)
    def zero_body(i):
      for r in range(16):
        out_acc[i, pl.ds(r * 16, 16)] = z16

    bufs = [rows_a, rows_b]
    sems = [sem_a, sem_b]

    def gather(k, b):
      return pltpu.make_async_copy(
          h_hbm.at[src_v.at[pl.ds(k * KG, KG)]], bufs[b], sems[b])

    @pl.when(nch > 0)
    def _():
      gather(0, 0).start()

    def compute(k, b):
      gather(k, b).wait()
      rows_v = bufs[b]

      @plsc.parallel_loop(0, KG // 16, 1)
      def grp_body(g):
        nrm16 = norm_v[pl.ds(k * KG + g * 16, 16)]
        dl16 = dstloc_v[pl.ds(k * KG + g * 16, 16)]
        for i in range(16):
          nrm = nrm16[i]
          dl = dl16[i]
          for r in range(16):
            sl = pl.ds(r * 16, 16)
            val = rows_v[g * 16 + i, sl] * nrm
            plsc.addupdate(out_acc.at[dl, sl], val)

    def pair_body(k2, _):
      for b in range(2):
        k = k2 * 2 + b

        @pl.when(k < nch)
        def _():
          @pl.when(k + 1 < nch)
          def _():
            gather(k + 1, 1 - b).start()

          compute(k, b)
      return 0

    lax.fori_loop(0, (nch + 1) // 2, pair_body, 0)
    pltpu.sync_copy(out_acc, out_hbm.at[pl.ds(wid * rpt, rpt)])

  run = pl.kernel(
      body,
      out_type=jax.ShapeDtypeStruct((np_, 256), F32),
      mesh=_mesh(),
      compiler_params=pltpu.CompilerParams(needs_layout_passes=False),
      scratch_types=[
          pltpu.VMEM((CAP,), I32),
          pltpu.VMEM((CAP,), I32),
          pltpu.VMEM((CAP,), F32),
          pltpu.VMEM((KG, 256), F32),
          pltpu.VMEM((KG, 256), F32),
          pltpu.VMEM((rpt, 256), F32),
          pltpu.VMEM((16,), I32),
          pltpu.SemaphoreType.DMA,
          pltpu.SemaphoreType.DMA,
      ],
  )
  return run(h, src_s, dstloc_s, norm_s, counts)


def _mm_body(relu_in, x_ref, w_ref, b_ref, o_ref):
  xb = x_ref[...]
  if relu_in:
    xb = jnp.maximum(xb, 0.0)
  o_ref[...] = jnp.dot(xb, w_ref[...], preferred_element_type=F32) + b_ref[...]


def _matmul(x, w, b, relu_in, blk=512):
  np_, d = x.shape
  h = w.shape[1]
  return pl.pallas_call(
      functools.partial(_mm_body, relu_in),
      grid=(np_ // blk,),
      in_specs=[
          pl.BlockSpec((blk, d), lambda i: (i, 0)),
          pl.BlockSpec((d, h), lambda i: (0, 0)),
          pl.BlockSpec((1, h), lambda i: (0, 0)),
      ],
      out_specs=pl.BlockSpec((blk, h), lambda i: (i, 0)),
      out_shape=jax.ShapeDtypeStruct((np_, h), F32),
  )(x, w, b.reshape(1, -1))


def _actor_head_body(na, x_ref, w_ref, b_ref, lp_ref, ent_ref):
  xb = jnp.maximum(x_ref[...], 0.0)
  lg = jnp.dot(xb, w_ref[...], preferred_element_type=F32) + b_ref[...]
  col = lax.broadcasted_iota(I32, lg.shape, 1)
  valid = col < na
  lgm = jnp.where(valid, lg, -1e30)
  m = jnp.max(lgm, axis=1, keepdims=True)
  e = jnp.exp(lgm - m)
  s = jnp.sum(e, axis=1, keepdims=True)
  p = e / s
  lp = jnp.log(jnp.maximum(p, 1e-12))
  lp_ref[...] = lp
  ent = -jnp.sum(jnp.where(valid, p * lp, 0.0), axis=1, keepdims=True)
  ent_ref[...] = jnp.broadcast_to(ent, ent_ref.shape)


def _actor_head(x, w_pad, b_pad, na, blk=512):
  np_, d = x.shape
  h = w_pad.shape[1]
  return pl.pallas_call(
      functools.partial(_actor_head_body, na),
      grid=(np_ // blk,),
      in_specs=[
          pl.BlockSpec((blk, d), lambda i: (i, 0)),
          pl.BlockSpec((d, h), lambda i: (0, 0)),
          pl.BlockSpec((1, h), lambda i: (0, 0)),
      ],
      out_specs=[
          pl.BlockSpec((blk, h), lambda i: (i, 0)),
          pl.BlockSpec((blk, h), lambda i: (i, 0)),
      ],
      out_shape=[
          jax.ShapeDtypeStruct((np_, h), F32),
          jax.ShapeDtypeStruct((np_, h), F32),
      ],
  )(x, w_pad, b_pad.reshape(1, -1))


def _pick(lp_full, ent_full, pidx, act, ng_pad):
  def body(lp_h, ent_h, pidx_h, act_h, alp_out, ent_out,
           pidx_v, act_v, lpr, entr, alp_v, ent_v):
    wid = _wid()

    @pl.when(wid == 0)
    def _():
      pltpu.sync_copy(pidx_h, pidx_v)
      pltpu.sync_copy(act_h, act_v)
      pltpu.sync_copy(lp_h.at[pidx_v], lpr)
      pltpu.sync_copy(ent_h.at[pidx_v], entr)
      base_iota = lax.iota(I32, 16)
      for j in range(ng_pad // 16):
        ri = base_iota + j * 16
        a16 = act_v[pl.ds(j * 16, 16)]
        alp_v[pl.ds(j * 16, 16)] = plsc.load_gather(lpr, [ri, a16])
        ent_v[pl.ds(j * 16, 16)] = plsc.load_gather(entr, [ri, ri * 0])
      pltpu.sync_copy(alp_v, alp_out)
      pltpu.sync_copy(ent_v, ent_out)

  run = pl.kernel(
      body,
      out_type=(
          jax.ShapeDtypeStruct((ng_pad,), F32),
          jax.ShapeDtypeStruct((ng_pad,), F32),
      ),
      mesh=_mesh(),
      compiler_params=pltpu.CompilerParams(needs_layout_passes=False),
      scratch_types=[
          pltpu.VMEM((ng_pad,), I32),
          pltpu.VMEM((ng_pad,), I32),
          pltpu.VMEM((ng_pad, 128), F32),
          pltpu.VMEM((ng_pad, 128), F32),
          pltpu.VMEM((ng_pad,), F32),
          pltpu.VMEM((ng_pad,), F32),
      ],
  )
  return run(lp_full, ent_full, pidx, act)


def kernel(x, edge_index, edge_attr, actions, ep, len_list_,
           Wb1, bb1, Wb2, bb2, Wb3, bb3, Wa1, ba1, Wa2, ba2, Wa_out, ba_out,
           Wc1, bc1, Wc2, bc2, Wc_out, bc_out):
  n, d = x.shape
  rpt = -(-n // NT)
  rpt = ((rpt + 15) // 16) * 16
  np_ = rpt * NT
  half = np_ // 2
  na = Wa_out.shape[1]
  ng = actions.shape[0]
  ng_pad = ((ng + 15) // 16) * 16

  src = edge_index[0]
  dst = edge_index[1]
  ew = edge_attr.reshape(-1).astype(F32)
  x_p = jnp.pad(x, ((0, np_ - n), (0, 0)))

  src_s, dstloc_s, norm_s, counts = _route_edges(src, dst, ew, n, rpt)

  def gcn(h_in, w, b, relu_in):
    return _aggregate(_matmul(h_in, w, b, relu_in), src_s, dstloc_s,
                      norm_s, counts, np_, rpt)

  g1 = gcn(x_p, Wb1, bb1, False)
  g2 = gcn(g1, Wb2, bb2, True)
  feat = gcn(g2, Wb3, bb3, True)

  gc1 = gcn(feat, Wc1, bc1, False)
  gc2 = gcn(gc1, Wc2, bc2, True)
  wc_pad = jnp.pad(Wc_out, ((0, 0), (0, 128 - Wc_out.shape[1])))
  bc_pad = jnp.pad(bc_out, (0, 128 - bc_out.shape[0]))
  values_full = _matmul(gc2, wc_pad, bc_pad, True)
  values = values_full[:n, :1]

  ga1 = gcn(feat, Wa1, ba1, False)
  ga2 = gcn(ga1, Wa2, ba2, True)
  wa_pad = jnp.pad(Wa_out, ((0, 0), (0, 128 - na)))
  ba_pad = jnp.pad(ba_out, (0, 128 - na))
  lp_full, ent_full = _actor_head(ga2, wa_pad, ba_pad, na)

  len_list = jnp.concatenate([jnp.zeros((1,), len_list_.dtype),
                              len_list_[:-1]])
  pidx = (len_list + ep).astype(I32)
  pidx = jnp.pad(pidx, (0, ng_pad - ng))
  act_pad = jnp.pad(actions.astype(I32), (0, ng_pad - ng))

  alp, ent = _pick(lp_full, ent_full, pidx, act_pad, ng_pad)
  return (alp[:ng], values, ent[:ng])


# speedup vs baseline: 1.1322x; 1.1322x over previous
import functools

import jax
import jax.numpy as jnp
from jax import lax
from jax.experimental import pallas as pl
from jax.experimental.pallas import tpu as pltpu
from jax.experimental.pallas import tpu_sc as plsc

F32 = jnp.float32
I32 = jnp.int32

NC = 2
NS = 16
NT = NC * NS
CAP = 6144
KG = 48


def _mesh():
  return plsc.VectorSubcoreMesh(core_axis_name="c", subcore_axis_name="s")


def _wid():
  return lax.axis_index("s") * NC + lax.axis_index("c")


def _route_edges(src, dst, ew, n_nodes, rpt):
  E = src.shape[0]
  chunk = 2000
  while E % chunk:
    chunk //= 2
  n_chunks = E // chunk
  iters = chunk // 16

  def body(src_hbm, dst_hbm, ew_hbm, src_s, dstloc_s, norm_s, counts,
           srcch, dstch, ewch, srcb, dstb, ewb, deg, cnt_v):
    wid = _wid()
    lo = wid * rpt

    def chunk_body(k, pos):
      off = k * chunk
      pltpu.sync_copy(src_hbm.at[pl.ds(off, chunk)], srcch)
      pltpu.sync_copy(dst_hbm.at[pl.ds(off, chunk)], dstch)
      pltpu.sync_copy(ew_hbm.at[pl.ds(off, chunk)], ewch)

      def inner(i, pos):
        s16 = srcch[pl.ds(i * 16, 16)]
        d16 = dstch[pl.ds(i * 16, 16)]
        e16 = ewch[pl.ds(i * 16, 16)]
        dl = d16 - lo
        mask = (dl >= 0) & (dl < rpt)
        cnt = plsc.all_reduce_population_count(mask)[0]
        plsc.store_compressed(srcb.at[pl.ds(pos, 16)], s16, mask=mask)
        plsc.store_compressed(dstb.at[pl.ds(pos, 16)], dl, mask=mask)
        plsc.store_compressed(ewb.at[pl.ds(pos, 16)], e16, mask=mask)
        return pos + cnt

      return lax.fori_loop(0, iters, inner, pos)

    pos = lax.fori_loop(0, n_chunks, chunk_body, jnp.int32(0))

    z16i = jnp.zeros((16,), I32)
    z16f = jnp.zeros((16,), F32)
    for t in range(4):
      srcb[pl.ds(pos + t * 16, 16)] = z16i
      dstb[pl.ds(pos + t * 16, 16)] = z16i
      ewb[pl.ds(pos + t * 16, 16)] = z16f
    cnt_p = ((pos + KG - 1) // KG) * KG

    for r in range(rpt // 16):
      deg[pl.ds(r * 16, 16)] = z16f

    def deg_body(i, _):
      d16 = dstb[pl.ds(i * 16, 16)]
      e16 = ewb[pl.ds(i * 16, 16)]
      plsc.addupdate_scatter(deg, [d16], e16)
      return 0

    lax.fori_loop(0, cnt_p // 16, deg_body, 0)

    def norm_body(i, _):
      d16 = dstb[pl.ds(i * 16, 16)]
      e16 = ewb[pl.ds(i * 16, 16)]
      dg = plsc.load_gather(deg, [d16])
      ewb[pl.ds(i * 16, 16)] = e16 / jnp.maximum(dg, 1e-6)
      return 0

    lax.fori_loop(0, cnt_p // 16, norm_body, 0)

    pltpu.sync_copy(srcb.at[pl.ds(0, CAP)], src_s.at[wid])
    pltpu.sync_copy(dstb.at[pl.ds(0, CAP)], dstloc_s.at[wid])
    pltpu.sync_copy(ewb.at[pl.ds(0, CAP)], norm_s.at[wid])
    cnt_v[...] = jnp.full((16,), cnt_p, I32)
    pltpu.sync_copy(cnt_v, counts.at[wid])

  run = pl.kernel(
      body,
      out_type=(
          jax.ShapeDtypeStruct((NT, CAP), I32),
          jax.ShapeDtypeStruct((NT, CAP), I32),
          jax.ShapeDtypeStruct((NT, CAP), F32),
          jax.ShapeDtypeStruct((NT, 16), I32),
      ),
      mesh=_mesh(),
      compiler_params=pltpu.CompilerParams(needs_layout_passes=False),
      scratch_types=[
          pltpu.VMEM((chunk,), I32),
          pltpu.VMEM((chunk,), I32),
          pltpu.VMEM((chunk,), F32),
          pltpu.VMEM((CAP + 64,), I32),
          pltpu.VMEM((CAP + 64,), I32),
          pltpu.VMEM((CAP + 64,), F32),
          pltpu.VMEM((rpt,), F32),
          pltpu.VMEM((16,), I32),
      ],
  )
  return run(src, dst, ew)


def _aggregate(h, src_s, dstloc_s, norm_s, counts, np_, rpt):
  acc_n = rpt * 256

  def body(h_hbm, src_s_h, dstloc_s_h, norm_s_h, counts_h, out_hbm,
           src_v, dstloc_v, norm_v, rows_a, rows_b, out_acc, cnt_v,
           sem_a, sem_b):
    wid = _wid()
    pltpu.sync_copy(src_s_h.at[wid], src_v)
    pltpu.sync_copy(dstloc_s_h.at[wid], dstloc_v)
    pltpu.sync_copy(norm_s_h.at[wid], norm_v)
    pltpu.sync_copy(counts_h.at[wid], cnt_v)
    cnt_p = cnt_v[pl.ds(0, 16)][0]
    nch = cnt_p // KG

    z16 = jnp.zeros((16,), F32)

    @plsc.parallel_loop(0, acc_n // 16, 1)
    def zero_body(i):
      out_acc[pl.ds(i * 16, 16)] = z16

    iota16 = lax.iota(I32, 16)
    bufs = [rows_a, rows_b]
    sems = [sem_a, sem_b]

    def gather(k, b):
      return pltpu.make_async_copy(
          h_hbm.at[src_v.at[pl.ds(k * KG, KG)]], bufs[b], sems[b])

    @pl.when(nch > 0)
    def _():
      gather(0, 0).start()

    def compute(k, b):
      gather(k, b).wait()
      rows_v = bufs[b]

      @plsc.parallel_loop(0, KG // 16, 1)
      def grp_body(g):
        nrm16 = norm_v[pl.ds(k * KG + g * 16, 16)]
        dl16 = dstloc_v[pl.ds(k * KG + g * 16, 16)]
        for i in range(16):
          nrm = nrm16[i]
          base = dl16[i] * 256
          for r in range(16):
            val = rows_v[g * 16 + i, pl.ds(r * 16, 16)] * nrm
            plsc.addupdate_scatter(out_acc, [iota16 + (base + r * 16)], val)

    def pair_body(k2, _):
      for b in range(2):
        k = k2 * 2 + b

        @pl.when(k < nch)
        def _():
          @pl.when(k + 1 < nch)
          def _():
            gather(k + 1, 1 - b).start()

          compute(k, b)
      return 0

    lax.fori_loop(0, (nch + 1) // 2, pair_body, 0)
    pltpu.sync_copy(out_acc, out_hbm.at[pl.ds(wid * acc_n, acc_n)])

  run = pl.kernel(
      body,
      out_type=jax.ShapeDtypeStruct((np_ * 256,), F32),
      mesh=_mesh(),
      compiler_params=pltpu.CompilerParams(needs_layout_passes=False),
      scratch_types=[
          pltpu.VMEM((CAP,), I32),
          pltpu.VMEM((CAP,), I32),
          pltpu.VMEM((CAP,), F32),
          pltpu.VMEM((KG, 256), F32),
          pltpu.VMEM((KG, 256), F32),
          pltpu.VMEM((acc_n,), F32),
          pltpu.VMEM((16,), I32),
          pltpu.SemaphoreType.DMA,
          pltpu.SemaphoreType.DMA,
      ],
  )
  return run(h, src_s, dstloc_s, norm_s, counts).reshape(np_, 256)


def _aggregate_masked(h, src_s, dstloc_s, norm_s, counts, pidx, np_, rpt):
  acc_n = rpt * 256
  mcap = 1024
  ng_pad = pidx.shape[0]

  def body(h_hbm, src_s_h, dstloc_s_h, norm_s_h, counts_h, pidx_h, out_hbm,
           src_v, dstloc_v, norm_v, rows_v, out_acc, marks,
           csrc, cdl, cnrm, pv, cnt_v):
    wid = _wid()
    lo = wid * rpt
    pltpu.sync_copy(src_s_h.at[wid], src_v)
    pltpu.sync_copy(dstloc_s_h.at[wid], dstloc_v)
    pltpu.sync_copy(norm_s_h.at[wid], norm_v)
    pltpu.sync_copy(counts_h.at[wid], cnt_v)
    pltpu.sync_copy(pidx_h, pv)
    cnt_p = cnt_v[pl.ds(0, 16)][0]

    z16 = jnp.zeros((16,), F32)

    @plsc.parallel_loop(0, acc_n // 16, 1)
    def zero_body(i):
      out_acc[pl.ds(i * 16, 16)] = z16

    for r in range(rpt // 16):
      marks[pl.ds(r * 16, 16)] = z16

    ones16 = jnp.ones((16,), F32)
    for j in range(ng_pad // 16):
      p16 = pv[pl.ds(j * 16, 16)]
      loc16 = p16 - lo
      mk = (loc16 >= 0) & (loc16 < rpt)
      plsc.store_scatter(marks, [jnp.where(mk, loc16, 0)], ones16, mask=mk)

    def filt_body(i, cpos):
      dl16 = dstloc_v[pl.ds(i * 16, 16)]
      s16 = src_v[pl.ds(i * 16, 16)]
      n16 = norm_v[pl.ds(i * 16, 16)]
      mk = plsc.load_gather(marks, [dl16]) > 0.0
      cnt = plsc.all_reduce_population_count(mk)[0]
      plsc.store_compressed(csrc.at[pl.ds(cpos, 16)], s16, mask=mk)
      plsc.store_compressed(cdl.at[pl.ds(cpos, 16)], dl16, mask=mk)
      plsc.store_compressed(cnrm.at[pl.ds(cpos, 16)], n16, mask=mk)
      return cpos + cnt

    cpos = lax.fori_loop(0, cnt_p // 16, filt_body, jnp.int32(0))

    z16i = jnp.zeros((16,), I32)
    for t in range(KG // 16):
      csrc[pl.ds(cpos + t * 16, 16)] = z16i
      cdl[pl.ds(cpos + t * 16, 16)] = z16i
      cnrm[pl.ds(cpos + t * 16, 16)] = z16
    cntp = ((cpos + KG - 1) // KG) * KG

    iota16 = lax.iota(I32, 16)

    def chunk_body(k, _):
      pltpu.sync_copy(h_hbm.at[csrc.at[pl.ds(k * KG, KG)]], rows_v)

      @plsc.parallel_loop(0, KG // 16, 1)
      def grp_body(g):
        nrm16 = cnrm[pl.ds(k * KG + g * 16, 16)]
        dl16 = cdl[pl.ds(k * KG + g * 16, 16)]
        for i in range(16):
          nrm = nrm16[i]
          base = dl16[i] * 256
          for r in range(16):
            val = rows_v[g * 16 + i, pl.ds(r * 16, 16)] * nrm
            plsc.addupdate_scatter(out_acc, [iota16 + (base + r * 16)], val)

      return 0

    lax.fori_loop(0, cntp // KG, chunk_body, 0)
    pltpu.sync_copy(out_acc, out_hbm.at[pl.ds(wid * acc_n, acc_n)])

  run = pl.kernel(
      body,
      out_type=jax.ShapeDtypeStruct((np_ * 256,), F32),
      mesh=_mesh(),
      compiler_params=pltpu.CompilerParams(needs_layout_passes=False),
      scratch_types=[
          pltpu.VMEM((CAP,), I32),
          pltpu.VMEM((CAP,), I32),
          pltpu.VMEM((CAP,), F32),
          pltpu.VMEM((KG, 256), F32),
          pltpu.VMEM((acc_n,), F32),
          pltpu.VMEM((rpt,), F32),
          pltpu.VMEM((mcap + KG,), I32),
          pltpu.VMEM((mcap + KG,), I32),
          pltpu.VMEM((mcap + KG,), F32),
          pltpu.VMEM((ng_pad,), I32),
          pltpu.VMEM((16,), I32),
      ],
  )
  return run(h, src_s, dstloc_s, norm_s, counts, pidx).reshape(np_, 256)


def _mm_body(relu_in, x_ref, w_ref, b_ref, o_ref):
  xb = x_ref[...]
  if relu_in:
    xb = jnp.maximum(xb, 0.0)
  o_ref[...] = jnp.dot(xb, w_ref[...], preferred_element_type=F32) + b_ref[...]


def _matmul(x, w, b, relu_in, blk=512):
  np_, d = x.shape
  h = w.shape[1]
  return pl.pallas_call(
      functools.partial(_mm_body, relu_in),
      grid=(np_ // blk,),
      in_specs=[
          pl.BlockSpec((blk, d), lambda i: (i, 0)),
          pl.BlockSpec((d, h), lambda i: (0, 0)),
          pl.BlockSpec((1, h), lambda i: (0, 0)),
      ],
      out_specs=pl.BlockSpec((blk, h), lambda i: (i, 0)),
      out_shape=jax.ShapeDtypeStruct((np_, h), F32),
  )(x, w, b.reshape(1, -1))


def _actor_head_body(na, x_ref, w_ref, b_ref, lp_ref, ent_ref):
  xb = jnp.maximum(x_ref[...], 0.0)
  lg = jnp.dot(xb, w_ref[...], preferred_element_type=F32) + b_ref[...]
  col = lax.broadcasted_iota(I32, lg.shape, 1)
  valid = col < na
  lgm = jnp.where(valid, lg, -1e30)
  m = jnp.max(lgm, axis=1, keepdims=True)
  e = jnp.exp(lgm - m)
  s = jnp.sum(e, axis=1, keepdims=True)
  p = e / s
  lp = jnp.log(jnp.maximum(p, 1e-12))
  lp_ref[...] = lp
  ent = -jnp.sum(jnp.where(valid, p * lp, 0.0), axis=1, keepdims=True)
  ent_ref[...] = jnp.broadcast_to(ent, ent_ref.shape)


def _actor_head(x, w_pad, b_pad, na, blk=512):
  np_, d = x.shape
  h = w_pad.shape[1]
  return pl.pallas_call(
      functools.partial(_actor_head_body, na),
      grid=(np_ // blk,),
      in_specs=[
          pl.BlockSpec((blk, d), lambda i: (i, 0)),
          pl.BlockSpec((d, h), lambda i: (0, 0)),
          pl.BlockSpec((1, h), lambda i: (0, 0)),
      ],
      out_specs=[
          pl.BlockSpec((blk, h), lambda i: (i, 0)),
          pl.BlockSpec((blk, h), lambda i: (i, 0)),
      ],
      out_shape=[
          jax.ShapeDtypeStruct((np_, h), F32),
          jax.ShapeDtypeStruct((np_, h), F32),
      ],
  )(x, w_pad, b_pad.reshape(1, -1))


def _pick(lp_full, ent_full, pidx, act, ng_pad):
  def body(lp_h, ent_h, pidx_h, act_h, alp_out, ent_out,
           pidx_v, act_v, lpr, entr, alp_v, ent_v):
    wid = _wid()

    @pl.when(wid == 0)
    def _():
      pltpu.sync_copy(pidx_h, pidx_v)
      pltpu.sync_copy(act_h, act_v)
      pltpu.sync_copy(lp_h.at[pidx_v], lpr)
      pltpu.sync_copy(ent_h.at[pidx_v], entr)
      base_iota = lax.iota(I32, 16)
      for j in range(ng_pad // 16):
        ri = base_iota + j * 16
        a16 = act_v[pl.ds(j * 16, 16)]
        alp_v[pl.ds(j * 16, 16)] = plsc.load_gather(lpr, [ri, a16])
        ent_v[pl.ds(j * 16, 16)] = plsc.load_gather(entr, [ri, ri * 0])
      pltpu.sync_copy(alp_v, alp_out)
      pltpu.sync_copy(ent_v, ent_out)

  run = pl.kernel(
      body,
      out_type=(
          jax.ShapeDtypeStruct((ng_pad,), F32),
          jax.ShapeDtypeStruct((ng_pad,), F32),
      ),
      mesh=_mesh(),
      compiler_params=pltpu.CompilerParams(needs_layout_passes=False),
      scratch_types=[
          pltpu.VMEM((ng_pad,), I32),
          pltpu.VMEM((ng_pad,), I32),
          pltpu.VMEM((ng_pad, 128), F32),
          pltpu.VMEM((ng_pad, 128), F32),
          pltpu.VMEM((ng_pad,), F32),
          pltpu.VMEM((ng_pad,), F32),
      ],
  )
  return run(lp_full, ent_full, pidx, act)


def kernel(x, edge_index, edge_attr, actions, ep, len_list_,
           Wb1, bb1, Wb2, bb2, Wb3, bb3, Wa1, ba1, Wa2, ba2, Wa_out, ba_out,
           Wc1, bc1, Wc2, bc2, Wc_out, bc_out):
  n, d = x.shape
  rpt = -(-n // NT)
  rpt = ((rpt + 15) // 16) * 16
  np_ = rpt * NT
  half = np_ // 2
  na = Wa_out.shape[1]
  ng = actions.shape[0]
  ng_pad = ((ng + 15) // 16) * 16

  src = edge_index[0]
  dst = edge_index[1]
  ew = edge_attr.reshape(-1).astype(F32)
  x_p = jnp.pad(x, ((0, np_ - n), (0, 0)))

  src_s, dstloc_s, norm_s, counts = _route_edges(src, dst, ew, n, rpt)

  def gcn(h_in, w, b, relu_in):
    return _aggregate(_matmul(h_in, w, b, relu_in), src_s, dstloc_s,
                      norm_s, counts, np_, rpt)

  g1 = gcn(x_p, Wb1, bb1, False)
  g2 = gcn(g1, Wb2, bb2, True)
  feat = gcn(g2, Wb3, bb3, True)

  gc1 = gcn(feat, Wc1, bc1, False)
  gc2 = gcn(gc1, Wc2, bc2, True)
  wc_pad = jnp.pad(Wc_out, ((0, 0), (0, 128 - Wc_out.shape[1])))
  bc_pad = jnp.pad(bc_out, (0, 128 - bc_out.shape[0]))
  values_full = _matmul(gc2, wc_pad, bc_pad, True)
  values = values_full[:n, :1]

  len_list = jnp.concatenate([jnp.zeros((1,), len_list_.dtype),
                              len_list_[:-1]])
  pidx = (len_list + ep).astype(I32)
  pidx = jnp.pad(pidx, (0, ng_pad - ng))
  act_pad = jnp.pad(actions.astype(I32), (0, ng_pad - ng))

  ga1 = gcn(feat, Wa1, ba1, False)
  qa = _matmul(ga1, Wa2, ba2, True)
  ga2 = _aggregate_masked(qa, src_s, dstloc_s, norm_s, counts, pidx,
                          np_, rpt)
  wa_pad = jnp.pad(Wa_out, ((0, 0), (0, 128 - na)))
  ba_pad = jnp.pad(ba_out, (0, 128 - na))
  lp_full, ent_full = _actor_head(ga2, wa_pad, ba_pad, na)

  alp, ent = _pick(lp_full, ent_full, pidx, act_pad, ng_pad)
  return (alp[:ng], values, ent[:ng])
